# Initial kernel scaffold; baseline (speedup 1.0000x reference)
import jax
import jax.numpy as jnp
from jax.experimental import pallas as pl


def kernel(nodes, to_neighs, id2feat, W, b):
    raise NotImplementedError("write your pallas kernel here")


# speedup vs baseline: 10.6692x; 10.6692x over previous
import functools

import jax
import jax.numpy as jnp
from jax import lax
from jax.experimental import pallas as pl
from jax.experimental.pallas import tpu as pltpu
from jax.experimental.pallas import tpu_sc as plsc

_NC = 2
_NS = 16
_NW = _NC * _NS
_LANES = 16
_NODES_PER_CHUNK = 4


def _sc_neighbor_sums(tn, feat, S):
    nw, nch, rows_per_chunk = tn.shape
    _, D = feat.shape
    npc = rows_per_chunk // S
    cpw = nch * npc
    B = nw * cpw
    dv = D // _LANES
    mesh = plsc.VectorSubcoreMesh(
        core_axis_name="c", subcore_axis_name="s",
        num_cores=_NC, num_subcores=_NS)

    @functools.partial(
        pl.kernel,
        out_type=jax.ShapeDtypeStruct((B, D), jnp.float32),
        mesh=mesh,
        scratch_types=[
            pltpu.VMEM((nch, rows_per_chunk), jnp.int32),
            pltpu.VMEM((2, rows_per_chunk, D), jnp.float32),
            pltpu.VMEM((cpw, D), jnp.float32),
            pltpu.SemaphoreType.DMA,
            pltpu.SemaphoreType.DMA,
        ],
    )
    def sums_kernel(tn_hbm, feat_hbm, out_hbm, idx_v, rows_v, out_v, sem0, sem1):
        wid = lax.axis_index("s") * _NC + lax.axis_index("c")
        pltpu.sync_copy(tn_hbm.at[wid], idx_v)
        pltpu.async_copy(feat_hbm.at[idx_v.at[0]], rows_v.at[0], sem0)
        pltpu.async_copy(feat_hbm.at[idx_v.at[1]], rows_v.at[1], sem1)
        sems = (sem0, sem1)

        def reduce_chunk(buf, c):
            for j in range(npc):
                def body(s, accs):
                    return tuple(
                        accs[d] + buf[j * S + s, pl.ds(d * _LANES, _LANES)]
                        for d in range(dv))
                accs = lax.fori_loop(
                    0, S, body,
                    tuple(jnp.zeros((_LANES,), jnp.float32) for _ in range(dv)),
                    unroll=4)
                for d in range(dv):
                    out_v[c * npc + j, pl.ds(d * _LANES, _LANES)] = accs[d]

        def pair(pc, carry):
            c0 = 2 * pc
            for k in range(2):
                pltpu.make_async_copy(
                    feat_hbm.at[idx_v.at[k]], rows_v.at[k], sems[k]).wait()
                reduce_chunk(rows_v.at[k], c0 + k)

                @pl.when(c0 + 2 + k < nch)
                def _():
                    pltpu.async_copy(
                        feat_hbm.at[idx_v.at[c0 + 2 + k]], rows_v.at[k], sems[k])
            return carry

        lax.fori_loop(0, nch // 2, pair, 0)
        pltpu.sync_copy(out_v, out_hbm.at[pl.ds(wid * cpw, cpw)])

    return sums_kernel(tn, feat)


def _tc_linear(x, W, b, S):
    B, D_IN = x.shape
    D_OUT = W.shape[1]
    blk = min(B, 2048)
    scale = 1.0 / S

    def body(x_ref, w_ref, b_ref, o_ref):
        o_ref[...] = (
            jnp.dot(x_ref[...], w_ref[...], preferred_element_type=jnp.float32)
            * scale + b_ref[...])

    return pl.pallas_call(
        body,
        grid=(B // blk,),
        in_specs=[
            pl.BlockSpec((blk, D_IN), lambda i: (i, 0)),
            pl.BlockSpec((D_IN, D_OUT), lambda i: (0, 0)),
            pl.BlockSpec((1, D_OUT), lambda i: (0, 0)),
        ],
        out_specs=pl.BlockSpec((blk, D_OUT), lambda i: (i, 0)),
        out_shape=jax.ShapeDtypeStruct((B, D_OUT), jnp.float32),
    )(x, W, b.reshape(1, D_OUT))


def kernel(nodes, to_neighs, id2feat, W, b):
    B, S = to_neighs.shape
    rows_per_chunk = _NODES_PER_CHUNK * S
    nch = B // (_NW * _NODES_PER_CHUNK)
    tn = to_neighs.astype(jnp.int32).reshape(_NW, nch, rows_per_chunk)
    sums = _sc_neighbor_sums(tn, id2feat, S)
    return _tc_linear(sums, W, b, S)
